# baseline (speedup 1.0000x reference)
import functools

import jax
import jax.numpy as jnp
from jax import lax
from jax.experimental import pallas as pl
from jax.experimental.pallas import tpu as pltpu
from jax.experimental.pallas import tpu_sc as plsc

_B = 4096
_L = 200
_D = 64
_V = 1000000
_NC = 2
_NS = 16
_NW = _NC * _NS
_ROWS = _B // _NW
_CHUNK = 100
_NCHUNK = _L // _CHUNK
_NV = _D // 16
_CSTRIDE = 104

_TCOLS = _V // 128
_VTAIL = _TCOLS * 128
_IROWS = _VTAIL + 128
_TPW = 245


def _transpose_body(tabt_hbm, tail_hbm, out_hbm,
                    src0, src1, dst0, dst1, tail_v,
                    sin0, sin1, sout0, sout1):
    wid = lax.axis_index("s") * _NC + lax.axis_index("c")

    rows_m = [lax.iota(jnp.int32, 16) + 16 * m for m in range(8)]

    def in_copy(c, src, sem):
        return pltpu.make_async_copy(
            tabt_hbm.at[:, pl.ds(c * 128, 128)], src, sem)

    def out_copy(c, dst, sem):
        return pltpu.make_async_copy(
            dst, out_hbm.at[pl.ds(c * 128, 128)], sem)

    def transpose_block(src, dst):
        @plsc.parallel_loop(0, _D, unroll=8)
        def _(d):
            col_d = jnp.full((16,), d, dtype=jnp.int32)
            for m in range(8):
                plsc.store_scatter(dst, [rows_m[m], col_d],
                                   src[d, pl.ds(16 * m, 16)])

    def block_of(i):
        return wid + _NW * i

    def stage(i, src, dst, sin, sout):
        c = block_of(i)

        @pl.when(c < _TCOLS)
        def _():
            in_copy(c, src, sin).wait()

            @pl.when(i >= 2)
            def _():
                out_copy(block_of(i - 2), dst, sout).wait()

            transpose_block(src, dst)
            out_copy(c, dst, sout).start()

            @pl.when(block_of(i + 2) < _TCOLS)
            def _():
                in_copy(block_of(i + 2), src, sin).start()

    @pl.when(block_of(0) < _TCOLS)
    def _():
        in_copy(block_of(0), src0, sin0).start()

    @pl.when(block_of(1) < _TCOLS)
    def _():
        in_copy(block_of(1), src1, sin1).start()

    def outer(k, carry):
        stage(2 * k, src0, dst0, sin0, sout0)
        stage(2 * k + 1, src1, dst1, sin1, sout1)
        return carry

    lax.fori_loop(0, (_TPW + 1) // 2, outer, 0)

    for i in (_TPW - 3, _TPW - 2, _TPW - 1):
        dst, sout = (dst0, sout0) if i % 2 == 0 else (dst1, sout1)

        @pl.when((block_of(i) < _TCOLS) & (block_of(i + 2) >= _TCOLS))
        def _(i=i, dst=dst, sout=sout):
            out_copy(block_of(i), dst, sout).wait()

    @pl.when(wid == 0)
    def _():
        pltpu.sync_copy(tail_hbm, tail_v)

        def body(v, carry):
            for k in range(_NV):
                dst0[v, pl.ds(16 * k, 16)] = tail_v[v, pl.ds(16 * k, 16)]
            return carry
        lax.fori_loop(0, 64, body, 0, unroll=4)
        pltpu.sync_copy(dst0.at[pl.ds(0, 64)],
                        out_hbm.at[pl.ds(_VTAIL, 64)])


def _sc_transpose(tabt, tail):
    mesh = plsc.VectorSubcoreMesh(core_axis_name="c", subcore_axis_name="s")
    return pl.kernel(
        _transpose_body,
        out_type=jax.ShapeDtypeStruct((_IROWS, 128), jnp.float32),
        mesh=mesh,
        scratch_types=[
            pltpu.VMEM((_D, 128), jnp.float32),
            pltpu.VMEM((_D, 128), jnp.float32),
            pltpu.VMEM((128, 128), jnp.float32),
            pltpu.VMEM((128, 128), jnp.float32),
            pltpu.VMEM((64, _D), jnp.float32),
            pltpu.SemaphoreType.DMA,
            pltpu.SemaphoreType.DMA,
            pltpu.SemaphoreType.DMA,
            pltpu.SemaphoreType.DMA,
        ],
        compiler_params=pltpu.CompilerParams(needs_layout_passes=False),
    )(tabt, tail)


def _pool_body(x_hbm, tab_hbm, out_hbm,
               idx_v, rows0, rows1, pooled_v, sem0, sem1):
    wid = lax.axis_index("s") * _NC + lax.axis_index("c")
    base = wid * _ROWS
    pltpu.sync_copy(x_hbm.at[pl.ds(base * _NCHUNK, _ROWS * _NCHUNK)], idx_v)

    def copies(r, rows_v, sem):
        return [
            pltpu.make_async_copy(
                tab_hbm.at[idx_v.at[r * _NCHUNK + c]],
                rows_v.at[pl.ds(c * _CSTRIDE, _CHUNK)],
                sem,
            )
            for c in range(_NCHUNK)
        ]

    def start(r, rows_v, sem):
        for cp in copies(r, rows_v, sem):
            cp.start()

    def wait(r, rows_v, sem):
        for cp in copies(r, rows_v, sem):
            cp.wait()

    inv_len = jnp.full((16,), 1.0 / _L, dtype=jnp.float32)

    def reduce_row(rows_v, r):
        accs = tuple(jnp.zeros((16,), jnp.float32) for _ in range(_NV))
        for c in range(_NCHUNK):
            def body(j, accs, _c=c):
                return tuple(
                    a + rows_v[_c * _CSTRIDE + j, pl.ds(v * 16, 16)]
                    for v, a in enumerate(accs))
            accs = lax.fori_loop(0, _CHUNK, body, accs, unroll=8)
        for v, a in enumerate(accs):
            pooled_v[r, pl.ds(v * 16, 16)] = a * inv_len

    start(0, rows0, sem0)

    def outer(k, carry):
        r0 = 2 * k
        wait(r0, rows0, sem0)
        start(r0 + 1, rows1, sem1)
        reduce_row(rows0, r0)
        wait(r0 + 1, rows1, sem1)

        @pl.when(r0 + 2 < _ROWS)
        def _():
            start(r0 + 2, rows0, sem0)

        reduce_row(rows1, r0 + 1)
        return carry

    lax.fori_loop(0, _ROWS // 2, outer, 0)
    pltpu.sync_copy(pooled_v, out_hbm.at[pl.ds(base, _ROWS)])


def _sc_pool(x2, table128):
    mesh = plsc.VectorSubcoreMesh(core_axis_name="c", subcore_axis_name="s")
    nbuf = (_NCHUNK - 1) * _CSTRIDE + _CHUNK
    return pl.kernel(
        _pool_body,
        out_type=jax.ShapeDtypeStruct((_B, _D), jnp.float32),
        mesh=mesh,
        scratch_types=[
            pltpu.VMEM((_ROWS * _NCHUNK, _CHUNK), jnp.int32),
            pltpu.VMEM((nbuf, 128), jnp.float32),
            pltpu.VMEM((nbuf, 128), jnp.float32),
            pltpu.VMEM((_ROWS, _D), jnp.float32),
            pltpu.SemaphoreType.DMA,
            pltpu.SemaphoreType.DMA,
        ],
    )(x2, table128)


def _mlp_body(p_ref, w1_ref, b1_ref, w2_ref, b2_ref, o_ref):
    h = jnp.dot(p_ref[...], w1_ref[...], preferred_element_type=jnp.float32)
    h = jnp.maximum(h + b1_ref[...], 0.0)
    o = jnp.dot(h, w2_ref[...], preferred_element_type=jnp.float32)
    o = o + b2_ref[...]
    m = jnp.max(o, axis=1, keepdims=True)
    lse = jnp.log(jnp.sum(jnp.exp(o - m), axis=1, keepdims=True)) + m
    o_ref[...] = o - lse


def _mlp(pooled, W1, b1, W2, b2):
    return pl.pallas_call(
        _mlp_body,
        out_shape=jax.ShapeDtypeStruct((_B, 2), jnp.float32),
    )(pooled, W1, b1, W2, b2)


def kernel(x, embedding_matrix, W1, b1, W2, b2):
    x2 = x.astype(jnp.int32).reshape(_B * _NCHUNK, _CHUNK)
    tabt = embedding_matrix.T
    tail = embedding_matrix[_VTAIL:]
    table128 = _sc_transpose(tabt, tail)
    pooled = _sc_pool(x2, table128)
    return _mlp(pooled, W1, b1.reshape(1, -1), W2, b2.reshape(1, -1))


# speedup vs baseline: 1.4206x; 1.4206x over previous
import functools

import jax
import jax.numpy as jnp
from jax import lax
from jax.experimental import pallas as pl
from jax.experimental.pallas import tpu as pltpu
from jax.experimental.pallas import tpu_sc as plsc

_B = 4096
_L = 200
_D = 64
_NC = 2
_NS = 16
_NW = _NC * _NS
_ROWS = _B // _NW
_CHUNK = 100
_NCHUNK = _L // _CHUNK
_NV = _D // 16


def _pool_body(x_hbm, tab_hbm, out_hbm,
               idx_v, rows0, rows1, pooled_v, sem0, sem1):
    wid = lax.axis_index("s") * _NC + lax.axis_index("c")
    base = wid * _ROWS
    pltpu.sync_copy(x_hbm.at[pl.ds(base * _NCHUNK, _ROWS * _NCHUNK)], idx_v)

    def chunk_copy(r, c, rows_v, sem):
        return pltpu.make_async_copy(
            tab_hbm.at[idx_v.at[r * _NCHUNK + c]], rows_v, sem)

    inv_len = jnp.full((16,), 1.0 / _L, dtype=jnp.float32)

    def reduce_row(rows_v, r):
        def body(j, accs):
            return tuple(a + rows_v[j, pl.ds(v * 16, 16)]
                         for v, a in enumerate(accs))
        accs = lax.fori_loop(
            0, _CHUNK, body,
            tuple(jnp.zeros((16,), jnp.float32) for _ in range(_NV)),
            unroll=8)
        for v, a in enumerate(accs):
            pooled_v[r, pl.ds(v * 16, 16)] = a * inv_len

    chunk_copy(0, 0, rows0, sem0).start()

    def outer(k, carry):
        r0 = 2 * k
        chunk_copy(r0, 0, rows0, sem0).wait()
        chunk_copy(r0, 1, rows0, sem0).start(add=True)

        @pl.when(r0 + 1 < _ROWS)
        def _():
            chunk_copy(r0 + 1, 0, rows1, sem1).start()

        chunk_copy(r0, 1, rows0, sem0).wait()
        reduce_row(rows0, r0)

        chunk_copy(r0 + 1, 0, rows1, sem1).wait()
        chunk_copy(r0 + 1, 1, rows1, sem1).start(add=True)

        @pl.when(r0 + 2 < _ROWS)
        def _():
            chunk_copy(r0 + 2, 0, rows0, sem0).start()

        chunk_copy(r0 + 1, 1, rows1, sem1).wait()
        reduce_row(rows1, r0 + 1)
        return carry

    lax.fori_loop(0, _ROWS // 2, outer, 0)
    pltpu.sync_copy(pooled_v, out_hbm.at[pl.ds(base, _ROWS)])


def _sc_pool(x2, table):
    mesh = plsc.VectorSubcoreMesh(core_axis_name="c", subcore_axis_name="s")
    return pl.kernel(
        _pool_body,
        out_type=jax.ShapeDtypeStruct((_B, _D), jnp.float32),
        mesh=mesh,
        scratch_types=[
            pltpu.VMEM((_ROWS * _NCHUNK, _CHUNK), jnp.int32),
            pltpu.VMEM((_CHUNK, _D), jnp.float32),
            pltpu.VMEM((_CHUNK, _D), jnp.float32),
            pltpu.VMEM((_ROWS, _D), jnp.float32),
            pltpu.SemaphoreType.DMA,
            pltpu.SemaphoreType.DMA,
        ],
        compiler_params=pltpu.CompilerParams(use_tc_tiling_on_sc=False),
    )(x2, table)


def _mlp_body(p_ref, w1_ref, b1_ref, w2_ref, b2_ref, o_ref):
    h = jnp.dot(p_ref[...], w1_ref[...], preferred_element_type=jnp.float32)
    h = jnp.maximum(h + b1_ref[...], 0.0)
    o = jnp.dot(h, w2_ref[...], preferred_element_type=jnp.float32)
    o = o + b2_ref[...]
    m = jnp.max(o, axis=1, keepdims=True)
    lse = jnp.log(jnp.sum(jnp.exp(o - m), axis=1, keepdims=True)) + m
    o_ref[...] = o - lse


def _mlp(pooled, W1, b1, W2, b2):
    return pl.pallas_call(
        _mlp_body,
        out_shape=jax.ShapeDtypeStruct((_B, 2), jnp.float32),
    )(pooled, W1, b1, W2, b2)


def kernel(x, embedding_matrix, W1, b1, W2, b2):
    x2 = x.astype(jnp.int32).reshape(_B * _NCHUNK, _CHUNK)
    pooled = _sc_pool(x2, embedding_matrix)
    return _mlp(pooled, W1, b1.reshape(1, -1), W2, b2.reshape(1, -1))


# speedup vs baseline: 1.4405x; 1.0140x over previous
import functools

import jax
import jax.numpy as jnp
from jax import lax
from jax.experimental import pallas as pl
from jax.experimental.pallas import tpu as pltpu
from jax.experimental.pallas import tpu_sc as plsc

_B = 4096
_L = 200
_D = 64
_NC = 2
_NS = 16
_NW = _NC * _NS
_ROWS = _B // _NW
_CHUNK = 100
_NCHUNK = _L // _CHUNK
_NV = _D // 16


def _pool_body(x_hbm, tab_hbm, out_hbm,
               idx_v, rows0, rows1, pooled_v, sem0, sem1):
    wid = lax.axis_index("s") * _NC + lax.axis_index("c")
    base = wid * _ROWS
    pltpu.sync_copy(x_hbm.at[pl.ds(base * _NCHUNK, _ROWS * _NCHUNK)], idx_v)

    def copies(r, rows_v, sem):
        return [
            pltpu.make_async_copy(
                tab_hbm.at[idx_v.at[r * _NCHUNK + c]],
                rows_v.at[pl.ds(c * _CHUNK, _CHUNK)],
                sem,
            )
            for c in range(_NCHUNK)
        ]

    def start(r, rows_v, sem):
        for cp in copies(r, rows_v, sem):
            cp.start()

    def wait(r, rows_v, sem):
        for cp in copies(r, rows_v, sem):
            cp.wait()

    inv_len = jnp.full((16,), 1.0 / _L, dtype=jnp.float32)

    def reduce_row(rows_v, r):
        def body(j, accs):
            return tuple(a + rows_v[j, pl.ds(v * 16, 16)]
                         for v, a in enumerate(accs))
        accs = lax.fori_loop(
            0, _L, body,
            tuple(jnp.zeros((16,), jnp.float32) for _ in range(_NV)),
            unroll=8)
        for v, a in enumerate(accs):
            pooled_v[r, pl.ds(v * 16, 16)] = a * inv_len

    start(0, rows0, sem0)

    def outer(k, carry):
        r0 = 2 * k
        wait(r0, rows0, sem0)
        start(r0 + 1, rows1, sem1)
        reduce_row(rows0, r0)
        wait(r0 + 1, rows1, sem1)

        @pl.when(r0 + 2 < _ROWS)
        def _():
            start(r0 + 2, rows0, sem0)

        reduce_row(rows1, r0 + 1)
        return carry

    lax.fori_loop(0, _ROWS // 2, outer, 0)
    pltpu.sync_copy(pooled_v, out_hbm.at[pl.ds(base, _ROWS)])


def _sc_pool(x2, table):
    mesh = plsc.VectorSubcoreMesh(core_axis_name="c", subcore_axis_name="s")
    return pl.kernel(
        _pool_body,
        out_type=jax.ShapeDtypeStruct((_B, _D), jnp.float32),
        mesh=mesh,
        scratch_types=[
            pltpu.VMEM((_ROWS * _NCHUNK, _CHUNK), jnp.int32),
            pltpu.VMEM((_L, _D), jnp.float32),
            pltpu.VMEM((_L, _D), jnp.float32),
            pltpu.VMEM((_ROWS, _D), jnp.float32),
            pltpu.SemaphoreType.DMA,
            pltpu.SemaphoreType.DMA,
        ],
        compiler_params=pltpu.CompilerParams(use_tc_tiling_on_sc=False),
    )(x2, table)


def _mlp_body(p_ref, w1_ref, b1_ref, w2_ref, b2_ref, o_ref):
    h = jnp.dot(p_ref[...], w1_ref[...], preferred_element_type=jnp.float32)
    h = jnp.maximum(h + b1_ref[...], 0.0)
    o = jnp.dot(h, w2_ref[...], preferred_element_type=jnp.float32)
    o = o + b2_ref[...]
    m = jnp.max(o, axis=1, keepdims=True)
    lse = jnp.log(jnp.sum(jnp.exp(o - m), axis=1, keepdims=True)) + m
    o_ref[...] = o - lse


def _mlp(pooled, W1, b1, W2, b2):
    return pl.pallas_call(
        _mlp_body,
        out_shape=jax.ShapeDtypeStruct((_B, 2), jnp.float32),
    )(pooled, W1, b1, W2, b2)


def kernel(x, embedding_matrix, W1, b1, W2, b2):
    x2 = x.astype(jnp.int32).reshape(_B * _NCHUNK, _CHUNK)
    pooled = _sc_pool(x2, embedding_matrix)
    return _mlp(pooled, W1, b1.reshape(1, -1), W2, b2.reshape(1, -1))
